# baseline (speedup 1.0000x reference)
import functools

import jax
import jax.numpy as jnp
from jax import lax
from jax.experimental import pallas as pl
from jax.experimental.pallas import tpu as pltpu
from jax.experimental.pallas import tpu_sc as plsc

N = 10000
E = 320000
D = 128
H = 32

NC = 2
NS = 16
NW = NC * NS
CH = 128
TOTCH = E // CH
K_HI, K_LO = 80, 78
HI_TILES = (TOTCH - NW * K_LO) // 2
C0_CHUNKS = HI_TILES * K_HI + (NS - HI_TILES) * K_LO
KMAX = K_HI
NPAD = 10240
STRIPE = NPAD // NS

_mesh = plsc.VectorSubcoreMesh(core_axis_name="c", subcore_axis_name="s")


def _build_sc_degrees():
    @functools.partial(
        pl.kernel,
        out_type=jax.ShapeDtypeStruct((NC, 2, NPAD, 16), jnp.float32),
        mesh=_mesh,
        compiler_params=pltpu.CompilerParams(use_tc_tiling_on_sc=False),
        scratch_types=[
            pltpu.VMEM((KMAX, CH), jnp.int32),
            pltpu.VMEM((KMAX, CH), jnp.int32),
            pltpu.VMEM((CH, 16), jnp.float32),
            pltpu.VMEM((STRIPE, 16), jnp.float32),
            pltpu.VMEM_SHARED((NPAD, 16), jnp.float32),
            pltpu.SemaphoreType.DMA,
        ],
    )
    def deg_kernel(row_h, col_h, out_h, idxr, idxc, ones, zb, acc, sem):
        c = lax.axis_index("c")
        s = lax.axis_index("s")

        def fill(i, _):
            ones[i, pl.ds(0, 16)] = jnp.ones((16,), jnp.float32)
            zb[i, pl.ds(0, 16)] = jnp.zeros((16,), jnp.float32)
            return 0

        lax.fori_loop(0, CH, fill, 0)

        def fillz(i, _):
            zb[i, pl.ds(0, 16)] = jnp.zeros((16,), jnp.float32)
            return 0

        lax.fori_loop(CH, STRIPE, fillz, 0)

        def run(K, base):
            pltpu.sync_copy(row_h.at[pl.ds(base, K)], idxr.at[pl.ds(0, K)])
            pltpu.sync_copy(col_h.at[pl.ds(base, K)], idxc.at[pl.ds(0, K)])

            def phase(idx, j):
                pltpu.sync_copy(zb, acc.at[pl.ds(s * STRIPE, STRIPE)])
                plsc.subcore_barrier()

                def sc_add(i):
                    return pltpu.make_async_copy(ones, acc.at[idx.at[i]], sem)

                W = 4

                def body(i, _):
                    @pl.when(i >= W)
                    def _():
                        sc_add(i - W).wait()

                    sc_add(i).start(add=True)
                    return 0

                lax.fori_loop(0, K, body, 0)

                def drain(i, _):
                    sc_add(i).wait()
                    return 0

                lax.fori_loop(K - W, K, drain, 0)
                plsc.subcore_barrier()
                pltpu.sync_copy(acc.at[pl.ds(s * STRIPE, STRIPE)],
                                out_h.at[c].at[j].at[pl.ds(s * STRIPE, STRIPE)])
                plsc.subcore_barrier()

            phase(idxr, 0)
            phase(idxc, 1)

        @pl.when(c == 0)
        def _():
            @pl.when(s < HI_TILES)
            def _():
                run(K_HI, s * K_HI)

            @pl.when(s >= HI_TILES)
            def _():
                run(K_LO, HI_TILES * K_HI + (s - HI_TILES) * K_LO)

        @pl.when(c == 1)
        def _():
            run(K_LO, C0_CHUNKS + s * K_LO)

    return deg_kernel


def _build_sc_edge_pass():
    @functools.partial(
        pl.kernel,
        out_type=jax.ShapeDtypeStruct((NC, NPAD, H), jnp.float32),
        mesh=_mesh,
        compiler_params=pltpu.CompilerParams(use_tc_tiling_on_sc=False,
                                             needs_layout_passes=False),
        scratch_types=[
            pltpu.VMEM((KMAX, CH), jnp.int32),
            pltpu.VMEM((KMAX, CH), jnp.int32),
            pltpu.VMEM((2, CH, H), jnp.bfloat16),
            pltpu.VMEM((2, CH, H), jnp.float32),
            pltpu.VMEM((STRIPE, H), jnp.float32),
            pltpu.VMEM_SHARED((NPAD, H), jnp.float32),
            pltpu.SemaphoreType.DMA,
            pltpu.SemaphoreType.DMA,
        ],
    )
    def edge_kernel(feat_h, row_h, col_h, out_h, idxr, idxc, gbuf, fbuf, zb,
                    acc, sem_g, sem_s):
        c = lax.axis_index("c")
        s = lax.axis_index("s")

        def fillz(i, _):
            zb[i, pl.ds(0, 16)] = jnp.zeros((16,), jnp.float32)
            zb[i, pl.ds(16, 16)] = jnp.zeros((16,), jnp.float32)
            return 0

        lax.fori_loop(0, STRIPE, fillz, 0)
        pltpu.sync_copy(zb, acc.at[pl.ds(s * STRIPE, STRIPE)])
        plsc.subcore_barrier()

        def gd(i, slot):
            return pltpu.make_async_copy(feat_h.at[idxc.at[i]],
                                         gbuf.at[slot], sem_g)

        def sd(i, slot):
            return pltpu.make_async_copy(fbuf.at[slot],
                                         acc.at[idxr.at[i]], sem_s)

        def convert(slot):
            g = gbuf.at[slot]
            f = fbuf.at[slot]
            UNR = 8

            def rows(r, _):
                for u in range(UNR):
                    rr = r * UNR + u
                    va, vb = plsc.unpack(g[rr, :],
                                         format=plsc.PackFormat.INTERLEAVED)
                    f[rr, pl.ds(0, 16)] = va
                    f[rr, pl.ds(16, 16)] = vb
                return 0

            lax.fori_loop(0, CH // UNR, rows, 0)

        def run(K, base):
            assert K % 2 == 0
            pltpu.sync_copy(row_h.at[pl.ds(base, K)], idxr.at[pl.ds(0, K)])
            pltpu.sync_copy(col_h.at[pl.ds(base, K)], idxc.at[pl.ds(0, K)])
            gd(0, 0).start()
            gd(1, 1).start()

            def half(j, slot):
                ci = 2 * j + slot

                @pl.when(ci >= 2)
                def _():
                    sd(ci - 2, slot).wait()

                gd(ci, slot).wait()
                convert(slot)

                @pl.when(ci + 2 < K)
                def _():
                    gd(ci + 2, slot).start()

                sd(ci, slot).start(add=True)

            def body(j, _):
                half(j, 0)
                half(j, 1)
                return 0

            lax.fori_loop(0, K // 2, body, 0)
            sd(K - 2, 0).wait()
            sd(K - 1, 1).wait()

        @pl.when(c == 0)
        def _():
            @pl.when(s < HI_TILES)
            def _():
                run(K_HI, s * K_HI)

            @pl.when(s >= HI_TILES)
            def _():
                run(K_LO, HI_TILES * K_HI + (s - HI_TILES) * K_LO)

        @pl.when(c == 1)
        def _():
            run(K_LO, C0_CHUNKS + s * K_LO)

        plsc.subcore_barrier()
        pltpu.sync_copy(acc.at[pl.ds(s * STRIPE, STRIPE)],
                        out_h.at[c].at[pl.ds(s * STRIPE, STRIPE)])

    return edge_kernel


def _dinv_body(deg_ref, out_ref):
    d = deg_ref[...]
    deg = d[0] + d[1]
    out_ref[...] = jnp.where(deg > 0.0,
                             lax.rsqrt(jnp.maximum(deg, 1e-12)), 0.0)


def _dense1_body(x_ref, w1_ref, ws1_ref, ds_ref, hs_ref, skip_ref):
    x = x_ref[...]
    h = jnp.dot(x, w1_ref[...], preferred_element_type=jnp.float32)
    hs_ref[...] = (h * ds_ref[...]).astype(jnp.bfloat16)
    skip_ref[...] = jnp.dot(x, ws1_ref[...], preferred_element_type=jnp.float32)


def _dense2_body(acc_ref, skip_ref, dd_ref, ds_ref, b1_ref, w2_ref, ws2_ref,
                 hs2_ref, skip2_ref):
    a = acc_ref[...]
    agg = (a[0] + a[1]) * dd_ref[...]
    h1 = jnp.maximum(agg + skip_ref[...] + b1_ref[...], 0.0)
    h2p = jnp.dot(h1, w2_ref[...], preferred_element_type=jnp.float32)
    hs2_ref[...] = (h2p * ds_ref[...]).astype(jnp.bfloat16)
    skip2_ref[...] = jnp.dot(h1, ws2_ref[...], preferred_element_type=jnp.float32)


def _final_body(acc_ref, skip_ref, dd_ref, b2_ref, wd1_ref, bd1_ref,
                wd2_ref, bd2_ref, out_ref, pool_ref, *, nblocks):
    i = pl.program_id(0)

    @pl.when(i == 0)
    def _():
        pool_ref[...] = jnp.zeros_like(pool_ref)

    a = acc_ref[...]
    agg = (a[0] + a[1]) * dd_ref[...]
    h2 = jnp.maximum(agg + skip_ref[...] + b2_ref[...], 0.0)
    pool_ref[...] += jnp.sum(h2, axis=0, keepdims=True)

    @pl.when(i == nblocks - 1)
    def _():
        p = pool_ref[...]
        o = jnp.dot(p, wd1_ref[...], preferred_element_type=jnp.float32) + bd1_ref[...]
        out_ref[...] = jnp.dot(o, wd2_ref[...], preferred_element_type=jnp.float32) + bd2_ref[...]


_MB = 1000
_NB = N // _MB


def kernel(x, edge_index, e, W1, Ws1, b1, W2, Ws2, b2, Wd1, bd1, Wd2, bd2):
    del e
    ei = edge_index.astype(jnp.int32)
    row3 = ei[0].reshape(TOTCH, CH)
    col3 = ei[1].reshape(TOTCH, CH)

    deg_parts = _build_sc_degrees()(row3, col3)
    deg4 = deg_parts.reshape(NC, 2, NPAD * 16 // 128, 128)

    dinv4 = pl.pallas_call(
        _dinv_body,
        out_shape=jax.ShapeDtypeStruct((2, NPAD * 16 // 128, 128), jnp.float32),
    )(deg4)
    dcols = dinv4.reshape(2, NPAD, 16)[:, :N, 0]
    dd_col = dcols[0].reshape(N, 1)
    ds_col = dcols[1].reshape(N, 1)

    swz = lambda w: jnp.stack([w[:, :16], w[:, 16:]], axis=-1).reshape(w.shape[0], H)
    W1s = swz(W1)
    W2s = swz(W2)

    full = lambda *shape: pl.BlockSpec(shape, lambda i: (0,) * len(shape))

    hs1, skip1 = pl.pallas_call(
        _dense1_body,
        grid=(_NB,),
        in_specs=[
            pl.BlockSpec((_MB, D), lambda i: (i, 0)),
            full(D, H),
            full(D, H),
            pl.BlockSpec((_MB, 1), lambda i: (i, 0)),
        ],
        out_specs=[
            pl.BlockSpec((_MB, H), lambda i: (i, 0)),
            pl.BlockSpec((_MB, H), lambda i: (i, 0)),
        ],
        out_shape=[
            jax.ShapeDtypeStruct((N, H), jnp.bfloat16),
            jax.ShapeDtypeStruct((N, H), jnp.float32),
        ],
    )(x, W1s, Ws1, ds_col)

    edge_pass = _build_sc_edge_pass()
    acc1 = edge_pass(hs1, row3, col3)

    hs2, skip2 = pl.pallas_call(
        _dense2_body,
        grid=(_NB,),
        in_specs=[
            pl.BlockSpec((NC, _MB, H), lambda i: (0, i, 0)),
            pl.BlockSpec((_MB, H), lambda i: (i, 0)),
            pl.BlockSpec((_MB, 1), lambda i: (i, 0)),
            pl.BlockSpec((_MB, 1), lambda i: (i, 0)),
            full(1, H),
            full(H, H),
            full(H, H),
        ],
        out_specs=[
            pl.BlockSpec((_MB, H), lambda i: (i, 0)),
            pl.BlockSpec((_MB, H), lambda i: (i, 0)),
        ],
        out_shape=[
            jax.ShapeDtypeStruct((N, H), jnp.bfloat16),
            jax.ShapeDtypeStruct((N, H), jnp.float32),
        ],
    )(acc1, skip1, dd_col, ds_col, b1.reshape(1, H), W2s, Ws2)

    acc2 = edge_pass(hs2, row3, col3)

    out = pl.pallas_call(
        functools.partial(_final_body, nblocks=_NB),
        grid=(_NB,),
        in_specs=[
            pl.BlockSpec((NC, _MB, H), lambda i: (0, i, 0)),
            pl.BlockSpec((_MB, H), lambda i: (i, 0)),
            pl.BlockSpec((_MB, 1), lambda i: (i, 0)),
            full(1, H),
            full(H, 24),
            full(1, 24),
            full(24, 1),
            full(1, 1),
        ],
        out_specs=pl.BlockSpec((1, 1), lambda i: (0, 0)),
        out_shape=jax.ShapeDtypeStruct((1, 1), jnp.float32),
        scratch_shapes=[pltpu.VMEM((1, H), jnp.float32)],
    )(acc2, skip2, dd_col, b2.reshape(1, H), Wd1, bd1.reshape(1, 24),
      Wd2, bd2.reshape(1, 1))

    return out


# speedup vs baseline: 1.0162x; 1.0162x over previous
import functools

import jax
import jax.numpy as jnp
from jax import lax
from jax.experimental import pallas as pl
from jax.experimental.pallas import tpu as pltpu
from jax.experimental.pallas import tpu_sc as plsc

N = 10000
E = 320000
D = 128
H = 32

NC = 2
NS = 16
NW = NC * NS
CH = 128
TOTCH = E // CH
K_HI, K_LO = 80, 78
HI_TILES = (TOTCH - NW * K_LO) // 2
C0_CHUNKS = HI_TILES * K_HI + (NS - HI_TILES) * K_LO
KMAX = K_HI
NPAD = 10240
STRIPE = NPAD // NS

_mesh = plsc.VectorSubcoreMesh(core_axis_name="c", subcore_axis_name="s")


def _build_sc_degrees():
    @functools.partial(
        pl.kernel,
        out_type=jax.ShapeDtypeStruct((NC, 2, NPAD, 16), jnp.float32),
        mesh=_mesh,
        compiler_params=pltpu.CompilerParams(use_tc_tiling_on_sc=False),
        scratch_types=[
            pltpu.VMEM((KMAX, CH), jnp.int32),
            pltpu.VMEM((KMAX, CH), jnp.int32),
            pltpu.VMEM((CH, 16), jnp.float32),
            pltpu.VMEM((STRIPE, 16), jnp.float32),
            pltpu.VMEM_SHARED((NPAD, 16), jnp.float32),
            pltpu.SemaphoreType.DMA,
        ],
    )
    def deg_kernel(row_h, col_h, out_h, idxr, idxc, ones, zb, acc, sem):
        c = lax.axis_index("c")
        s = lax.axis_index("s")

        def fill(i, _):
            ones[i, pl.ds(0, 16)] = jnp.ones((16,), jnp.float32)
            zb[i, pl.ds(0, 16)] = jnp.zeros((16,), jnp.float32)
            return 0

        lax.fori_loop(0, CH, fill, 0)

        def fillz(i, _):
            zb[i, pl.ds(0, 16)] = jnp.zeros((16,), jnp.float32)
            return 0

        lax.fori_loop(CH, STRIPE, fillz, 0)

        def run(K, base):
            pltpu.sync_copy(row_h.at[pl.ds(base, K)], idxr.at[pl.ds(0, K)])
            pltpu.sync_copy(col_h.at[pl.ds(base, K)], idxc.at[pl.ds(0, K)])

            def phase(idx, j):
                pltpu.sync_copy(zb, acc.at[pl.ds(s * STRIPE, STRIPE)])
                plsc.subcore_barrier()

                def sc_add(i):
                    return pltpu.make_async_copy(ones, acc.at[idx.at[i]], sem)

                W = 4

                def body(i, _):
                    @pl.when(i >= W)
                    def _():
                        sc_add(i - W).wait()

                    sc_add(i).start(add=True)
                    return 0

                lax.fori_loop(0, K, body, 0)

                def drain(i, _):
                    sc_add(i).wait()
                    return 0

                lax.fori_loop(K - W, K, drain, 0)
                plsc.subcore_barrier()
                pltpu.sync_copy(acc.at[pl.ds(s * STRIPE, STRIPE)],
                                out_h.at[c].at[j].at[pl.ds(s * STRIPE, STRIPE)])
                plsc.subcore_barrier()

            phase(idxr, 0)
            phase(idxc, 1)

        @pl.when(c == 0)
        def _():
            @pl.when(s < HI_TILES)
            def _():
                run(K_HI, s * K_HI)

            @pl.when(s >= HI_TILES)
            def _():
                run(K_LO, HI_TILES * K_HI + (s - HI_TILES) * K_LO)

        @pl.when(c == 1)
        def _():
            run(K_LO, C0_CHUNKS + s * K_LO)

    return deg_kernel


def _build_sc_edge_pass():
    @functools.partial(
        pl.kernel,
        out_type=jax.ShapeDtypeStruct((NC, NPAD, H), jnp.float32),
        mesh=_mesh,
        compiler_params=pltpu.CompilerParams(use_tc_tiling_on_sc=False,
                                             needs_layout_passes=False),
        scratch_types=[
            pltpu.VMEM((KMAX, CH), jnp.int32),
            pltpu.VMEM((KMAX, CH), jnp.int32),
            pltpu.VMEM((2, CH, H), jnp.bfloat16),
            pltpu.VMEM((2, CH, H), jnp.float32),
            pltpu.VMEM((STRIPE, H), jnp.float32),
            pltpu.VMEM_SHARED((NPAD, H), jnp.float32),
            pltpu.SemaphoreType.DMA,
            pltpu.SemaphoreType.DMA,
        ],
    )
    def edge_kernel(feat_h, row_h, col_h, out_h, idxr, idxc, gbuf, fbuf, zb,
                    acc, sem_g, sem_s):
        c = lax.axis_index("c")
        s = lax.axis_index("s")

        def fillz(i, _):
            zb[i, pl.ds(0, 16)] = jnp.zeros((16,), jnp.float32)
            zb[i, pl.ds(16, 16)] = jnp.zeros((16,), jnp.float32)
            return 0

        lax.fori_loop(0, STRIPE, fillz, 0)
        pltpu.sync_copy(zb, acc.at[pl.ds(s * STRIPE, STRIPE)])
        plsc.subcore_barrier()

        def gd(i, slot):
            return pltpu.make_async_copy(feat_h.at[idxc.at[i]],
                                         gbuf.at[slot], sem_g)

        def sd(i, slot):
            return pltpu.make_async_copy(fbuf.at[slot],
                                         acc.at[idxr.at[i]], sem_s)

        def convert(slot):
            g = gbuf.at[slot]
            f = fbuf.at[slot]
            UNR = 4

            def rows(r, _):
                for u in range(UNR):
                    rr = r * UNR + u
                    va, vb = plsc.unpack(g[rr, :],
                                         format=plsc.PackFormat.INTERLEAVED)
                    f[rr, pl.ds(0, 16)] = va
                    f[rr, pl.ds(16, 16)] = vb
                return 0

            lax.fori_loop(0, CH // UNR, rows, 0)

        def run(K, base):
            assert K % 2 == 0
            pltpu.sync_copy(row_h.at[pl.ds(base, K)], idxr.at[pl.ds(0, K)])
            pltpu.sync_copy(col_h.at[pl.ds(base, K)], idxc.at[pl.ds(0, K)])
            gd(0, 0).start()
            gd(1, 1).start()

            def half(j, slot):
                ci = 2 * j + slot

                @pl.when(ci >= 2)
                def _():
                    sd(ci - 2, slot).wait()

                gd(ci, slot).wait()
                convert(slot)

                @pl.when(ci + 2 < K)
                def _():
                    gd(ci + 2, slot).start()

                sd(ci, slot).start(add=True)

            def body(j, _):
                half(j, 0)
                half(j, 1)
                return 0

            lax.fori_loop(0, K // 2, body, 0)
            sd(K - 2, 0).wait()
            sd(K - 1, 1).wait()

        @pl.when(c == 0)
        def _():
            @pl.when(s < HI_TILES)
            def _():
                run(K_HI, s * K_HI)

            @pl.when(s >= HI_TILES)
            def _():
                run(K_LO, HI_TILES * K_HI + (s - HI_TILES) * K_LO)

        @pl.when(c == 1)
        def _():
            run(K_LO, C0_CHUNKS + s * K_LO)

        plsc.subcore_barrier()
        pltpu.sync_copy(acc.at[pl.ds(s * STRIPE, STRIPE)],
                        out_h.at[c].at[pl.ds(s * STRIPE, STRIPE)])

    return edge_kernel


def _dinv_body(deg_ref, out_ref):
    d = deg_ref[...]
    deg = d[0] + d[1]
    out_ref[...] = jnp.where(deg > 0.0,
                             lax.rsqrt(jnp.maximum(deg, 1e-12)), 0.0)


def _matmul1_body(x_ref, w1_ref, ws1_ref, h_ref, skip_ref):
    x = x_ref[...]
    h_ref[...] = jnp.dot(x, w1_ref[...], preferred_element_type=jnp.float32)
    skip_ref[...] = jnp.dot(x, ws1_ref[...], preferred_element_type=jnp.float32)


def _scale1_body(h_ref, ds_ref, hs_ref):
    hs_ref[...] = (h_ref[...] * ds_ref[...]).astype(jnp.bfloat16)


def _dense2_body(acc_ref, skip_ref, dd_ref, ds_ref, b1_ref, w2_ref, ws2_ref,
                 hs2_ref, skip2_ref):
    a = acc_ref[...]
    agg = (a[0] + a[1]) * dd_ref[...]
    h1 = jnp.maximum(agg + skip_ref[...] + b1_ref[...], 0.0)
    h2p = jnp.dot(h1, w2_ref[...], preferred_element_type=jnp.float32)
    hs2_ref[...] = (h2p * ds_ref[...]).astype(jnp.bfloat16)
    skip2_ref[...] = jnp.dot(h1, ws2_ref[...], preferred_element_type=jnp.float32)


def _final_body(acc_ref, skip_ref, dd_ref, b2_ref, wd1_ref, bd1_ref,
                wd2_ref, bd2_ref, out_ref, pool_ref, *, nblocks):
    i = pl.program_id(0)

    @pl.when(i == 0)
    def _():
        pool_ref[...] = jnp.zeros_like(pool_ref)

    a = acc_ref[...]
    agg = (a[0] + a[1]) * dd_ref[...]
    h2 = jnp.maximum(agg + skip_ref[...] + b2_ref[...], 0.0)
    pool_ref[...] += jnp.sum(h2, axis=0, keepdims=True)

    @pl.when(i == nblocks - 1)
    def _():
        p = pool_ref[...]
        o = jnp.dot(p, wd1_ref[...], preferred_element_type=jnp.float32) + bd1_ref[...]
        out_ref[...] = jnp.dot(o, wd2_ref[...], preferred_element_type=jnp.float32) + bd2_ref[...]


_MB = 1000
_NB = N // _MB


def kernel(x, edge_index, e, W1, Ws1, b1, W2, Ws2, b2, Wd1, bd1, Wd2, bd2):
    del e
    ei = edge_index.astype(jnp.int32)
    row3 = ei[0].reshape(TOTCH, CH)
    col3 = ei[1].reshape(TOTCH, CH)

    deg_parts = _build_sc_degrees()(row3, col3)
    deg4 = deg_parts.reshape(NC, 2, NPAD * 16 // 128, 128)

    dinv4 = pl.pallas_call(
        _dinv_body,
        out_shape=jax.ShapeDtypeStruct((2, NPAD * 16 // 128, 128), jnp.float32),
    )(deg4)
    dcols = dinv4.reshape(2, NPAD, 16)[:, :N, 0]
    dd_col = dcols[0].reshape(N, 1)
    ds_col = dcols[1].reshape(N, 1)

    swz = lambda w: jnp.stack([w[:, :16], w[:, 16:]], axis=-1).reshape(w.shape[0], H)
    W1s = swz(W1)
    W2s = swz(W2)

    full = lambda *shape: pl.BlockSpec(shape, lambda i: (0,) * len(shape))

    hraw, skip1 = pl.pallas_call(
        _matmul1_body,
        grid=(_NB,),
        in_specs=[
            pl.BlockSpec((_MB, D), lambda i: (i, 0)),
            full(D, H),
            full(D, H),
        ],
        out_specs=[
            pl.BlockSpec((_MB, H), lambda i: (i, 0)),
            pl.BlockSpec((_MB, H), lambda i: (i, 0)),
        ],
        out_shape=[
            jax.ShapeDtypeStruct((N, H), jnp.float32),
            jax.ShapeDtypeStruct((N, H), jnp.float32),
        ],
    )(x, W1s, Ws1)

    hs1 = pl.pallas_call(
        _scale1_body,
        grid=(_NB,),
        in_specs=[
            pl.BlockSpec((_MB, H), lambda i: (i, 0)),
            pl.BlockSpec((_MB, 1), lambda i: (i, 0)),
        ],
        out_specs=pl.BlockSpec((_MB, H), lambda i: (i, 0)),
        out_shape=jax.ShapeDtypeStruct((N, H), jnp.bfloat16),
    )(hraw, ds_col)

    edge_pass = _build_sc_edge_pass()
    acc1 = edge_pass(hs1, row3, col3)

    hs2, skip2 = pl.pallas_call(
        _dense2_body,
        grid=(_NB,),
        in_specs=[
            pl.BlockSpec((NC, _MB, H), lambda i: (0, i, 0)),
            pl.BlockSpec((_MB, H), lambda i: (i, 0)),
            pl.BlockSpec((_MB, 1), lambda i: (i, 0)),
            pl.BlockSpec((_MB, 1), lambda i: (i, 0)),
            full(1, H),
            full(H, H),
            full(H, H),
        ],
        out_specs=[
            pl.BlockSpec((_MB, H), lambda i: (i, 0)),
            pl.BlockSpec((_MB, H), lambda i: (i, 0)),
        ],
        out_shape=[
            jax.ShapeDtypeStruct((N, H), jnp.bfloat16),
            jax.ShapeDtypeStruct((N, H), jnp.float32),
        ],
    )(acc1, skip1, dd_col, ds_col, b1.reshape(1, H), W2s, Ws2)

    acc2 = edge_pass(hs2, row3, col3)

    out = pl.pallas_call(
        functools.partial(_final_body, nblocks=_NB),
        grid=(_NB,),
        in_specs=[
            pl.BlockSpec((NC, _MB, H), lambda i: (0, i, 0)),
            pl.BlockSpec((_MB, H), lambda i: (i, 0)),
            pl.BlockSpec((_MB, 1), lambda i: (i, 0)),
            full(1, H),
            full(H, 24),
            full(1, 24),
            full(24, 1),
            full(1, 1),
        ],
        out_specs=pl.BlockSpec((1, 1), lambda i: (0, 0)),
        out_shape=jax.ShapeDtypeStruct((1, 1), jnp.float32),
        scratch_shapes=[pltpu.VMEM((1, H), jnp.float32)],
    )(acc2, skip2, dd_col, b2.reshape(1, H), Wd1, bd1.reshape(1, 24),
      Wd2, bd2.reshape(1, 1))

    return out
